# Initial kernel scaffold; baseline (speedup 1.0000x reference)
import jax
import jax.numpy as jnp
from jax.experimental import pallas as pl


def kernel(p, p_gs, f, group_idx, W_spe1, g_spe1, b_spe1, W_spe2, g_spe2, b_spe2, W_spe3, g_spe3, b_spe3, W_up0, W_dn0, g_res0, b_res0, W_up1, W_dn1, g_res1, b_res1, W_up2, W_dn2, g_res2, b_res2, W_up3, W_dn3, g_res3, b_res3, g_dec, b_dec, W_dec):
    raise NotImplementedError("write your pallas kernel here")


# speedup vs baseline: 2.2661x; 2.2661x over previous
import functools

import jax
import jax.numpy as jnp
from jax import lax
from jax.experimental import pallas as pl
from jax.experimental.pallas import tpu as pltpu
from jax.experimental.pallas import tpu_sc as plsc

N = 50000
K = 16
NP = 51200
TP = 256
GRID = NP // TP
EB = TP * K
NE = NP * K
M_EDGES = float(N * K)
M_PTS = float(N)

_f32 = jnp.float32
_i32 = jnp.int32

_NC = 2
_NS = 16
_NW = _NC * _NS

_EPW = NE // _NW
_HCH = 3200
_HNCH = _EPW // _HCH


def _sc_head_gather(tab_hbm, idx_hbm, out_hbm, idx_v, rows_v, sem):
    wid = lax.axis_index("s") * _NC + lax.axis_index("c")
    base = wid * _EPW

    def body(ch, _):
        off = base + ch * _HCH
        pltpu.sync_copy(idx_hbm.at[pl.ds(off, _HCH)], idx_v)
        pltpu.async_copy(tab_hbm.at[idx_v], rows_v, sem).wait()
        pltpu.sync_copy(rows_v, out_hbm.at[pl.ds(off, _HCH)])
        return _

    lax.fori_loop(0, _HNCH, body, None)


_SC_PARAMS = pltpu.CompilerParams(use_tc_tiling_on_sc=False)

_head_gather = functools.partial(
    pl.kernel,
    out_type=jax.ShapeDtypeStruct((NE, 8), _f32),
    mesh=plsc.VectorSubcoreMesh(core_axis_name="c", subcore_axis_name="s"),
    compiler_params=_SC_PARAMS,
    scratch_types=[
        pltpu.VMEM((_HCH,), _i32),
        pltpu.VMEM((_HCH, 8), _f32),
        pltpu.SemaphoreType.DMA,
    ],
)(_sc_head_gather)


_PPW = NP // _NW
_PC = 32
_BNCH = _PPW // _PC


def _sc_block_aggr(x_hbm, idx_hbm, out_hbm, idx_v, rows_v, out_v, sem):
    wid = lax.axis_index("s") * _NC + lax.axis_index("c")

    def chunk(ch, _):
        pbase = wid * _PPW + ch * _PC
        pltpu.sync_copy(idx_hbm.at[pl.ds(pbase * K, _PC * K)], idx_v)
        pltpu.async_copy(x_hbm.at[idx_v], rows_v, sem).wait()

        def point(pp, _):
            r0 = pp * K
            acc = [rows_v[r0, pl.ds(q * 16, 16)] for q in range(4)]
            for r in range(1, K):
                for q in range(4):
                    acc[q] = jnp.maximum(acc[q], rows_v[r0 + r, pl.ds(q * 16, 16)])
            for q in range(4):
                out_v[pp, pl.ds(q * 16, 16)] = acc[q]
            return _

        lax.fori_loop(0, _PC, point, None)
        pltpu.sync_copy(out_v, out_hbm.at[pl.ds(pbase, _PC)])
        return _

    lax.fori_loop(0, _BNCH, chunk, None)


_block_aggr = functools.partial(
    pl.kernel,
    out_type=jax.ShapeDtypeStruct((NP, 64), _f32),
    mesh=plsc.VectorSubcoreMesh(core_axis_name="c", subcore_axis_name="s"),
    compiler_params=_SC_PARAMS,
    scratch_types=[
        pltpu.VMEM((_PC * K,), _i32),
        pltpu.VMEM((_PC * K, 64), _f32),
        pltpu.VMEM((_PC, 64), _f32),
        pltpu.SemaphoreType.DMA,
    ],
)(_sc_block_aggr)


def _dot_head(a, b):
    return jnp.dot(a, b, preferred_element_type=_f32)


def _dot(a, b):
    return jnp.dot(a, b, preferred_element_type=_f32)


def _edge_mask(i):
    rowp = lax.broadcasted_iota(_i32, (EB, 1), 0) // K + i * TP
    return (rowp < N).astype(_f32)


def _point_mask(i):
    rowp = lax.broadcasted_iota(_i32, (TP, 1), 0) + i * TP
    return (rowp < N).astype(_f32)


def _accum_stats(oref, i, z, m):
    s = jnp.sum(z * m, axis=0)
    ss = jnp.sum(z * z * m, axis=0)
    c = s.shape[0]
    pad = jnp.zeros((1, 128 - c), _f32)
    srow = jnp.concatenate([s[None, :], pad], axis=1)
    ssrow = jnp.concatenate([ss[None, :], pad], axis=1)

    @pl.when(i == 0)
    def _():
        oref[...] = jnp.concatenate(
            [srow, ssrow, jnp.zeros((6, 128), _f32)], axis=0)

    @pl.when(i != 0)
    def _():
        cur = oref[0:2, :]
        comp = oref[2:4, :]
        y = jnp.concatenate([srow, ssrow], axis=0) - comp
        t = cur + y
        oref[2:4, :] = (t - cur) - y
        oref[0:2, :] = t


def _edge_chain(gref, pref, *ws):
    g = gref[...]
    pb = pref[...]
    colm = (lax.broadcasted_iota(_i32, (1, 8), 1) < 3).astype(_f32)
    e = (g.reshape(TP, K, 8) - (pb * colm)[:, None, :]).reshape(EB, 8)
    z = _dot_head(e, ws[0])
    if len(ws) == 1:
        return z
    s1 = ws[1]
    h = jnp.maximum(((z - s1[0:1, :32]) * s1[1:2, :32]) * s1[2:3, :32] + s1[3:4, :32], 0.0)
    z = _dot_head(h, ws[2])
    if len(ws) == 3:
        return z
    s2 = ws[3]
    h = jnp.maximum(((z - s2[0:1, :16]) * s2[1:2, :16]) * s2[2:3, :16] + s2[3:4, :16], 0.0)
    z = _dot_head(h, ws[4])
    return z


def _hstats1(gref, pref, w1ref, oref):
    i = pl.program_id(0)
    z = _edge_chain(gref, pref, w1ref[...])
    _accum_stats(oref, i, z, _edge_mask(i))


def _hstats2(gref, pref, w1ref, s1ref, w2ref, oref):
    i = pl.program_id(0)
    s1 = s1ref[...]
    z = _edge_chain(gref, pref, w1ref[...], s1, w2ref[...])
    _accum_stats(oref, i, z, _edge_mask(i))


def _hstats3(gref, pref, w1ref, s1ref, w2ref, s2ref, w3ref, oref):
    i = pl.program_id(0)
    s1 = s1ref[...]
    s2 = s2ref[...]
    z = _edge_chain(gref, pref, w1ref[...], s1, w2ref[...],
                    s2, w3ref[...])
    _accum_stats(oref, i, z, _edge_mask(i))


def _hfinal(gref, pref, w1ref, s1ref, w2ref, s2ref, w3ref, s3ref, oref):
    s1 = s1ref[...]
    s2 = s2ref[...]
    s3 = s3ref[...]
    z = _edge_chain(gref, pref, w1ref[...], s1, w2ref[...],
                    s2, w3ref[...])
    h3 = (((z - s3[0:1, :64]) * s3[1:2, :64]) * s3[2:3, :64] + s3[3:4, :64]).reshape(TP, K, 64)
    acc = h3[:, 0, :]
    for r in range(1, K):
        acc = jnp.maximum(acc, h3[:, r, :])
    oref[...] = acc


def _bstats(nref, wuref, wdref, oref):
    i = pl.program_id(0)
    a = jnp.maximum(_dot(nref[...], wuref[...]), 0.0)
    z = _dot(a, wdref[...])
    _accum_stats(oref, i, z, _point_mask(i))


def _bupdate(nref, xref, wuref, wdref, sref, oref):
    a = jnp.maximum(_dot(nref[...], wuref[...]), 0.0)
    z = _dot(a, wdref[...])
    s = sref[...]
    oref[...] = xref[...] + (((z - s[0:1, :64]) * s[1:2, :64]) * s[2:3, :64] + s[3:4, :64])


def _bupdate_stats(nref, xref, wuref, wdref, sref, oref, o2ref):
    i = pl.program_id(0)
    a = jnp.maximum(_dot(nref[...], wuref[...]), 0.0)
    z = _dot(a, wdref[...])
    s = sref[...]
    xn = xref[...] + (((z - s[0:1, :64]) * s[1:2, :64]) * s[2:3, :64] + s[3:4, :64])
    oref[...] = xn
    _accum_stats(o2ref, i, xn, _point_mask(i))


def _decoder(xref, sref, wref, oref):
    s = sref[...]
    xb = ((xref[...] - s[0:1, :64]) * s[1:2, :64]) * s[2:3, :64] + s[3:4, :64]
    oref[...] = _dot(xb, wref[...])


def _full(shape):
    return pl.BlockSpec(shape, lambda i: (0, 0))


_EDGE_BS = pl.BlockSpec((EB, 8), lambda i: (i, 0))
_PT8_BS = pl.BlockSpec((TP, 8), lambda i: (i, 0))
_PT64_BS = pl.BlockSpec((TP, 64), lambda i: (i, 0))
_TC_PARAMS = pltpu.CompilerParams(dimension_semantics=("arbitrary",))


def _stats_call(kfn, nout, in_specs, *args):
    del nout
    return pl.pallas_call(
        kfn,
        grid=(GRID,),
        in_specs=in_specs,
        out_specs=_full((8, 128)),
        out_shape=jax.ShapeDtypeStruct((8, 128), _f32),
        compiler_params=_TC_PARAMS,
    )(*args)


def _fold(stats, m, g, b):
    nc = g.shape[0]
    mu = (stats[0, :nc] - stats[2, :nc]) / m
    var = (stats[1, :nc] - stats[3, :nc]) / m - mu * mu
    rs = lax.rsqrt(var + 1e-5)
    return (jnp.zeros((8, 128), _f32)
            .at[0, :nc].set(mu).at[1, :nc].set(rs)
            .at[2, :nc].set(g).at[3, :nc].set(b))


def kernel(p, p_gs, f, group_idx, W_spe1, g_spe1, b_spe1, W_spe2, g_spe2, b_spe2,
           W_spe3, g_spe3, b_spe3, W_up0, W_dn0, g_res0, b_res0, W_up1, W_dn1,
           g_res1, b_res1, W_up2, W_dn2, g_res2, b_res2, W_up3, W_dn3, g_res3,
           b_res3, g_dec, b_dec, W_dec):
    tab = jnp.zeros((NP, 8), _f32)
    tab = tab.at[:N, :3].set(p).at[:N, 3:7].set(f)
    gidx = jnp.zeros((NP, K), _i32).at[:N].set(group_idx).reshape(-1)
    W1p = jnp.zeros((8, 32), _f32).at[:7].set(W_spe1)

    g = _head_gather(tab, gidx)

    st1 = _stats_call(_hstats1, 32, [_EDGE_BS, _PT8_BS, _full((8, 32))],
                      g, tab, W1p)
    f1 = _fold(st1, M_EDGES, g_spe1, b_spe1)

    st2 = _stats_call(_hstats2, 16,
                      [_EDGE_BS, _PT8_BS, _full((8, 32)), _full((8, 128)),
                       _full((32, 16))],
                      g, tab, W1p, f1, W_spe2)
    f2 = _fold(st2, M_EDGES, g_spe2, b_spe2)

    st3 = _stats_call(_hstats3, 64,
                      [_EDGE_BS, _PT8_BS, _full((8, 32)), _full((8, 128)),
                       _full((32, 16)), _full((8, 128)), _full((16, 64))],
                      g, tab, W1p, f1, W_spe2, f2, W_spe3)
    f3 = _fold(st3, M_EDGES, g_spe3, b_spe3)

    x = pl.pallas_call(
        _hfinal,
        grid=(GRID,),
        in_specs=[_EDGE_BS, _PT8_BS, _full((8, 32)), _full((8, 128)),
                  _full((32, 16)), _full((8, 128)), _full((16, 64)),
                  _full((8, 128))],
        out_specs=_PT64_BS,
        out_shape=jax.ShapeDtypeStruct((NP, 64), _f32),
        compiler_params=_TC_PARAMS,
    )(g, tab, W1p, f1, W_spe2, f2, W_spe3, f3)

    blocks = [(W_up0, W_dn0, g_res0, b_res0), (W_up1, W_dn1, g_res1, b_res1),
              (W_up2, W_dn2, g_res2, b_res2), (W_up3, W_dn3, g_res3, b_res3)]
    xstats = None
    for bi, (W_up, W_dn, gr, br) in enumerate(blocks):
        nbr = _block_aggr(x, gidx)
        stz = _stats_call(_bstats, 64, [_PT64_BS, _full((64, 128)),
                                        _full((128, 64))], nbr, W_up, W_dn)
        fz = _fold(stz, M_PTS, gr, br)
        if bi < 3:
            x = pl.pallas_call(
                _bupdate,
                grid=(GRID,),
                in_specs=[_PT64_BS, _PT64_BS, _full((64, 128)),
                          _full((128, 64)), _full((8, 128))],
                out_specs=_PT64_BS,
                out_shape=jax.ShapeDtypeStruct((NP, 64), _f32),
                compiler_params=_TC_PARAMS,
            )(nbr, x, W_up, W_dn, fz)
        else:
            x, xstats = pl.pallas_call(
                _bupdate_stats,
                grid=(GRID,),
                in_specs=[_PT64_BS, _PT64_BS, _full((64, 128)),
                          _full((128, 64)), _full((8, 128))],
                out_specs=[_PT64_BS, _full((8, 128))],
                out_shape=[jax.ShapeDtypeStruct((NP, 64), _f32),
                           jax.ShapeDtypeStruct((8, 128), _f32)],
                compiler_params=_TC_PARAMS,
            )(nbr, x, W_up, W_dn, fz)

    fd = _fold(xstats, M_PTS, g_dec, b_dec)

    out = pl.pallas_call(
        _decoder,
        grid=(GRID,),
        in_specs=[_PT64_BS, _full((8, 128)), _full((64, 256))],
        out_specs=pl.BlockSpec((TP, 256), lambda i: (i, 0)),
        out_shape=jax.ShapeDtypeStruct((NP, 256), _f32),
        compiler_params=_TC_PARAMS,
    )(x, fd, W_dec)
    return out[:N]
